# baseline (speedup 1.0000x reference)
import functools

import numpy as np
import jax
import jax.numpy as jnp
from jax import lax
from jax.experimental import pallas as pl
from jax.experimental.pallas import tpu as pltpu
from jax.experimental.pallas import tpu_sc as plsc

D_MODEL = 128
MAX_LEN = 200
IDXW = 100
NBUF = 3
NUM_WORKERS = 32


def _positional_encoding(max_len, d_model):
    pos = jnp.arange(max_len, dtype=jnp.float32)[:, None]
    div = jnp.exp(
        jnp.arange(0, d_model, 2, dtype=jnp.float32)
        * (-(np.log(10000.0)) / d_model)
    )
    ang = pos * div
    pe = jnp.zeros((max_len, d_model), dtype=jnp.float32)
    pe = pe.at[:, 0::2].set(jnp.sin(ang))
    pe = pe.at[:, 1::2].set(jnp.cos(ang))
    return pe


@functools.lru_cache(maxsize=None)
def _make_kernel(batch, seq_len):
    seqs_per_w = batch // NUM_WORKERS
    halves = seq_len // IDXW
    idx_rows = seqs_per_w * halves

    mesh = plsc.VectorSubcoreMesh(core_axis_name="c", subcore_axis_name="s")

    @functools.partial(
        pl.kernel,
        mesh=mesh,
        out_type=jax.ShapeDtypeStruct((batch, seq_len, D_MODEL), jnp.float32),
        scratch_types=[
            pltpu.VMEM((idx_rows, IDXW), jnp.int32),
            pltpu.VMEM((seq_len, D_MODEL), jnp.float32),
            pltpu.VMEM((NBUF, seq_len, D_MODEL), jnp.float32),
            pltpu.SemaphoreType.DMA((NBUF,)),
            pltpu.SemaphoreType.DMA((NBUF,)),
        ],
    )
    def k(idx_hbm, table_hbm, pe_hbm, out_hbm, idx_v, pe_v, rows_v, sem_g, sem_w):
        wid = lax.axis_index("s") * 2 + lax.axis_index("c")
        bbase = wid * seqs_per_w
        pltpu.sync_copy(pe_hbm, pe_v)
        pltpu.sync_copy(idx_hbm.at[pl.ds(wid * idx_rows, idx_rows)], idx_v)

        def gather_seq(c, b):
            for h in range(halves):
                pltpu.async_copy(
                    table_hbm.at[idx_v.at[halves * c + h]],
                    rows_v.at[b, pl.ds(h * IDXW, IDXW)],
                    sem_g.at[b],
                )

        gather_seq(0, 0)
        gather_seq(1, 1)

        def tri_body(p, carry):
            for b in range(NBUF):
                c = NBUF * p + b

                @pl.when(c < seqs_per_w)
                def _():
                    n = c + 2
                    nb = (b + 2) % NBUF

                    @pl.when(n < seqs_per_w)
                    def _():
                        @pl.when(n >= NBUF)
                        def _():
                            pltpu.make_async_copy(
                                rows_v.at[nb], out_hbm.at[0], sem_w.at[nb]
                            ).wait()

                        gather_seq(n, nb)

                    pltpu.make_async_copy(
                        out_hbm.at[0], rows_v.at[b], sem_g.at[b]
                    ).wait()

                    def row_body(r, carry2):
                        for j in range(D_MODEL // 16):
                            sl = pl.ds(j * 16, 16)
                            plsc.addupdate(rows_v.at[b, r, sl], pe_v[r, sl])
                        return carry2

                    lax.fori_loop(0, seq_len, row_body, 0)
                    pltpu.async_copy(
                        rows_v.at[b], out_hbm.at[bbase + c], sem_w.at[b]
                    )
            return carry

        lax.fori_loop(0, (seqs_per_w + NBUF - 1) // NBUF, tri_body, 0)

        for b in range(NBUF):
            pltpu.make_async_copy(
                rows_v.at[b], out_hbm.at[0], sem_w.at[b]
            ).wait()

    return k


def kernel(x, table):
    batch, seq_len = x.shape
    idx = x.reshape(-1, IDXW).astype(jnp.int32)
    pe = _positional_encoding(MAX_LEN, D_MODEL)[:seq_len]
    k = _make_kernel(batch, seq_len)
    return k(idx, table, pe)


# speedup vs baseline: 1.0068x; 1.0068x over previous
import functools

import numpy as np
import jax
import jax.numpy as jnp
from jax import lax
from jax.experimental import pallas as pl
from jax.experimental.pallas import tpu as pltpu
from jax.experimental.pallas import tpu_sc as plsc

D_MODEL = 128
MAX_LEN = 200
IDXW = 100
NBUF = 3
NUM_WORKERS = 32


def _positional_encoding(max_len, d_model):
    pos = np.arange(max_len, dtype=np.float32)[:, None]
    div = np.exp(
        np.arange(0, d_model, 2, dtype=np.float32)
        * (-(np.log(10000.0)) / d_model)
    )
    ang = pos * div
    pe = np.zeros((max_len, d_model), dtype=np.float32)
    pe[:, 0::2] = np.sin(ang)
    pe[:, 1::2] = np.cos(ang)
    return pe


@functools.lru_cache(maxsize=None)
def _make_kernel(batch, seq_len):
    seqs_per_w = batch // NUM_WORKERS
    halves = seq_len // IDXW

    mesh = plsc.VectorSubcoreMesh(core_axis_name="c", subcore_axis_name="s")

    @functools.partial(
        pl.kernel,
        mesh=mesh,
        out_type=jax.ShapeDtypeStruct((batch, seq_len, D_MODEL), jnp.float32),
        scratch_types=[
            pltpu.VMEM((seqs_per_w * halves, IDXW), jnp.int32),
            pltpu.VMEM((seq_len, D_MODEL), jnp.float32),
            pltpu.VMEM((NBUF, seq_len, D_MODEL), jnp.float32),
            pltpu.SemaphoreType.DMA((NBUF,)),
            pltpu.SemaphoreType.DMA((NBUF,)),
        ],
    )
    def k(idx_hbm, table_hbm, pe_hbm, out_hbm, idx_v, pe_v, rows_v, sem_g, sem_w):
        wid = lax.axis_index("s") * 2 + lax.axis_index("c")
        bbase = wid * seqs_per_w
        pltpu.sync_copy(pe_hbm, pe_v)
        pltpu.sync_copy(
            idx_hbm.at[pl.ds(bbase * halves, seqs_per_w * halves)], idx_v
        )

        def gather_seq(c, b):
            for h in range(halves):
                pltpu.async_copy(
                    table_hbm.at[idx_v.at[halves * c + h]],
                    rows_v.at[b, pl.ds(h * IDXW, IDXW)],
                    sem_g.at[b],
                )

        gather_seq(0, 0)
        gather_seq(1, 1)

        def tri_body(p, carry):
            for b in range(NBUF):
                c = NBUF * p + b

                @pl.when(c < seqs_per_w)
                def _():
                    n = c + 2
                    nb = (b + 2) % NBUF

                    @pl.when(n < seqs_per_w)
                    def _():
                        @pl.when(n >= NBUF)
                        def _():
                            pltpu.make_async_copy(
                                rows_v.at[nb], out_hbm.at[0], sem_w.at[nb]
                            ).wait()

                        gather_seq(n, nb)

                    pltpu.make_async_copy(
                        out_hbm.at[0], rows_v.at[b], sem_g.at[b]
                    ).wait()

                    def row_body(r, carry2):
                        for j in range(D_MODEL // 16):
                            sl = pl.ds(j * 16, 16)
                            plsc.addupdate(rows_v.at[b, r, sl], pe_v[r, sl])
                        return carry2

                    lax.fori_loop(0, seq_len, row_body, 0)
                    pltpu.async_copy(
                        rows_v.at[b], out_hbm.at[bbase + c], sem_w.at[b]
                    )
            return carry

        lax.fori_loop(0, (seqs_per_w + NBUF - 1) // NBUF, tri_body, 0)

        for b in range(NBUF):
            pltpu.make_async_copy(
                rows_v.at[b], out_hbm.at[0], sem_w.at[b]
            ).wait()

    return k


_PE = _positional_encoding(MAX_LEN, D_MODEL)


def kernel(x, table):
    batch, seq_len = x.shape
    idx = x.reshape(-1, IDXW).astype(jnp.int32)
    pe = jnp.asarray(_PE[:seq_len])
    k = _make_kernel(batch, seq_len)
    return k(idx, table, pe)


# speedup vs baseline: 1.1861x; 1.1781x over previous
import functools

import numpy as np
import jax
import jax.numpy as jnp
from jax import lax
from jax.experimental import pallas as pl
from jax.experimental.pallas import tpu as pltpu
from jax.experimental.pallas import tpu_sc as plsc

D_MODEL = 128
MAX_LEN = 200
IDXW = 100
NBUF = 3
NUM_WORKERS = 32


def _positional_encoding(max_len, d_model):
    pos = np.arange(max_len, dtype=np.float32)[:, None]
    div = np.exp(
        np.arange(0, d_model, 2, dtype=np.float32)
        * (-(np.log(10000.0)) / d_model)
    )
    ang = pos * div
    pe = np.zeros((max_len, d_model), dtype=np.float32)
    pe[:, 0::2] = np.sin(ang)
    pe[:, 1::2] = np.cos(ang)
    return pe


@functools.lru_cache(maxsize=None)
def _make_kernel(batch, seq_len):
    seqs_per_w = batch // NUM_WORKERS
    halves = seq_len // IDXW

    mesh = plsc.VectorSubcoreMesh(core_axis_name="c", subcore_axis_name="s")

    @functools.partial(
        pl.kernel,
        mesh=mesh,
        out_type=jax.ShapeDtypeStruct((batch, seq_len, D_MODEL), jnp.float32),
        scratch_types=[
            pltpu.VMEM((seqs_per_w * halves, IDXW), jnp.int32),
            pltpu.VMEM_SHARED((seq_len, D_MODEL), jnp.float32),
            pltpu.VMEM((NBUF, seq_len, D_MODEL), jnp.float32),
            pltpu.SemaphoreType.DMA((NBUF,)),
            pltpu.SemaphoreType.DMA((NBUF,)),
            pltpu.SemaphoreType.DMA((NBUF,)),
        ],
    )
    def k(idx_hbm, table_hbm, pe_hbm, out_hbm, idx_v, pe_v, rows_v, sem_g, sem_w, sem_p):
        wid = lax.axis_index("s") * 2 + lax.axis_index("c")
        bbase = wid * seqs_per_w

        @pl.when(lax.axis_index("s") == 0)
        def _():
            pltpu.sync_copy(pe_hbm, pe_v)

        plsc.subcore_barrier()
        pltpu.sync_copy(
            idx_hbm.at[pl.ds(bbase * halves, seqs_per_w * halves)], idx_v
        )

        def prep_seq(c, b):
            pltpu.async_copy(pe_v, rows_v.at[b], sem_p.at[b])
            pltpu.make_async_copy(pe_v, rows_v.at[b], sem_p.at[b]).wait()
            for h in range(halves):
                pltpu.async_copy(
                    table_hbm.at[idx_v.at[halves * c + h]],
                    rows_v.at[b, pl.ds(h * IDXW, IDXW)],
                    sem_g.at[b],
                    add=True,
                )

        prep_seq(0, 0)
        prep_seq(1, 1)

        def tri_body(p, carry):
            for b in range(NBUF):
                c = NBUF * p + b

                @pl.when(c < seqs_per_w)
                def _():
                    n = c + 2
                    nb = (b + 2) % NBUF

                    @pl.when(n < seqs_per_w)
                    def _():
                        @pl.when(n >= NBUF)
                        def _():
                            pltpu.make_async_copy(
                                rows_v.at[nb], out_hbm.at[0], sem_w.at[nb]
                            ).wait()

                        prep_seq(n, nb)

                    pltpu.make_async_copy(
                        out_hbm.at[0], rows_v.at[b], sem_g.at[b]
                    ).wait()

                    pltpu.async_copy(
                        rows_v.at[b], out_hbm.at[bbase + c], sem_w.at[b]
                    )
            return carry

        lax.fori_loop(0, (seqs_per_w + NBUF - 1) // NBUF, tri_body, 0)

        for b in range(NBUF):
            pltpu.make_async_copy(
                rows_v.at[b], out_hbm.at[0], sem_w.at[b]
            ).wait()

    return k


_PE = _positional_encoding(MAX_LEN, D_MODEL)


def kernel(x, table):
    batch, seq_len = x.shape
    idx = x.reshape(-1, IDXW).astype(jnp.int32)
    pe = jnp.asarray(_PE[:seq_len])
    k = _make_kernel(batch, seq_len)
    return k(idx, table, pe)


# speedup vs baseline: 1.2224x; 1.0306x over previous
import functools

import numpy as np
import jax
import jax.numpy as jnp
from jax import lax
from jax.experimental import pallas as pl
from jax.experimental.pallas import tpu as pltpu
from jax.experimental.pallas import tpu_sc as plsc

D_MODEL = 128
MAX_LEN = 200
IDXW = 100
NBUF = 4
NUM_WORKERS = 32


def _positional_encoding(max_len, d_model):
    pos = np.arange(max_len, dtype=np.float32)[:, None]
    div = np.exp(
        np.arange(0, d_model, 2, dtype=np.float32)
        * (-(np.log(10000.0)) / d_model)
    )
    ang = pos * div
    pe = np.zeros((max_len, d_model), dtype=np.float32)
    pe[:, 0::2] = np.sin(ang)
    pe[:, 1::2] = np.cos(ang)
    return pe


@functools.lru_cache(maxsize=None)
def _make_kernel(batch, seq_len):
    seqs_per_w = batch // NUM_WORKERS
    halves = seq_len // IDXW

    mesh = plsc.VectorSubcoreMesh(core_axis_name="c", subcore_axis_name="s")

    @functools.partial(
        pl.kernel,
        mesh=mesh,
        out_type=jax.ShapeDtypeStruct((batch, seq_len, D_MODEL), jnp.float32),
        scratch_types=[
            pltpu.VMEM((seqs_per_w * halves, IDXW), jnp.int32),
            pltpu.VMEM_SHARED((seq_len, D_MODEL), jnp.float32),
            pltpu.VMEM((NBUF, seq_len, D_MODEL), jnp.float32),
            pltpu.SemaphoreType.DMA((NBUF,)),
            pltpu.SemaphoreType.DMA((NBUF,)),
            pltpu.SemaphoreType.DMA((NBUF,)),
        ],
    )
    def k(idx_hbm, table_hbm, pe_hbm, out_hbm, idx_v, pe_v, rows_v, sem_g, sem_w, sem_p):
        wid = lax.axis_index("s") * 2 + lax.axis_index("c")
        bbase = wid * seqs_per_w

        @pl.when(lax.axis_index("s") == 0)
        def _():
            pltpu.sync_copy(pe_hbm, pe_v)

        plsc.subcore_barrier()
        pltpu.sync_copy(
            idx_hbm.at[pl.ds(bbase * halves, seqs_per_w * halves)], idx_v
        )

        def prefill(b):
            pltpu.async_copy(pe_v, rows_v.at[b], sem_p.at[b])

        def gather_seq(c, b):
            for h in range(halves):
                pltpu.async_copy(
                    table_hbm.at[idx_v.at[halves * c + h]],
                    rows_v.at[b, pl.ds(h * IDXW, IDXW)],
                    sem_g.at[b],
                    add=True,
                )

        def wait_prefill(b):
            pltpu.make_async_copy(pe_v, rows_v.at[b], sem_p.at[b]).wait()

        for b in range(3):
            prefill(b)
        for b in range(2):
            wait_prefill(b)
            gather_seq(b, b)

        def quad_body(p, carry):
            for b in range(NBUF):
                c = NBUF * p + b

                n3 = c + 3
                s3 = (b + 3) % NBUF

                @pl.when(n3 < seqs_per_w)
                def _():
                    @pl.when(n3 >= NBUF)
                    def _():
                        pltpu.make_async_copy(
                            rows_v.at[s3], out_hbm.at[0], sem_w.at[s3]
                        ).wait()

                    prefill(s3)

                n2 = c + 2
                s2 = (b + 2) % NBUF

                @pl.when(n2 < seqs_per_w)
                def _():
                    wait_prefill(s2)
                    gather_seq(n2, s2)

                pltpu.make_async_copy(
                    out_hbm.at[0], rows_v.at[b], sem_g.at[b]
                ).wait()
                pltpu.async_copy(
                    rows_v.at[b], out_hbm.at[bbase + c], sem_w.at[b]
                )
            return carry

        lax.fori_loop(0, seqs_per_w // NBUF, quad_body, 0)

        for b in range(NBUF):
            pltpu.make_async_copy(
                rows_v.at[b], out_hbm.at[0], sem_w.at[b]
            ).wait()

    return k


_PE = _positional_encoding(MAX_LEN, D_MODEL)


def kernel(x, table):
    batch, seq_len = x.shape
    idx = x.reshape(-1, IDXW).astype(jnp.int32)
    pe = jnp.asarray(_PE[:seq_len])
    k = _make_kernel(batch, seq_len)
    return k(idx, table, pe)
